# Initial kernel scaffold; baseline (speedup 1.0000x reference)
import jax
import jax.numpy as jnp
from jax.experimental import pallas as pl


def kernel(node_x, edge_index, edge_attr, global_feats, params):
    raise NotImplementedError("write your pallas kernel here")


# speedup vs baseline: 1.5218x; 1.5218x over previous
import functools

import jax
import jax.numpy as jnp
from jax import lax
from jax.experimental import pallas as pl
from jax.experimental.pallas import tpu as pltpu
from jax.experimental.pallas import tpu_sc as plsc

_N = 10000
_E = 320000
_NF = 64
_H = 64
_RN = 30
_RE = 10
_TW = 128

_NC, _NS = 2, 16
_NW = _NC * _NS
_CHUNK_E = 128
_ROWS = 2560
_EP = _ROWS * 128
_CPW = _ROWS // _NW
_NR = _CPW // 2
_HALF_N = 5000
_ACC_ROWS = 5120
_TPR = _ACC_ROWS // _NS
_CPT = _ROWS // _NS


@functools.cache
def _sc_mesh():
    return plsc.VectorSubcoreMesh(
        core_axis_name="c", subcore_axis_name="s",
        num_cores=_NC, num_subcores=_NS)


def _gather_body(t_hbm, idx_hbm, gr_hbm, gc_hbm,
                 idx0, idx1, br0, bc0, br1, bc1,
                 semi0, semi1, semg0, semg1, semo0, semo1):
    sid = lax.axis_index("s")
    wid = sid * _NC + lax.axis_index("c")
    base = wid * _CPW
    slots = ((idx0, br0, bc0, semi0, semg0, semo0),
             (idx1, br1, bc1, semi1, semg1, semo1))

    def idx_start(g, s):
        (idx, _, _, semi, _, _) = s
        pltpu.async_copy(idx_hbm.at[pl.ds(g, 1)], idx, semi)

    def idx_wait(g, s):
        (idx, _, _, semi, _, _) = s
        pltpu.make_async_copy(idx_hbm.at[pl.ds(g, 1)], idx, semi).wait()

    def gathers_start(s):
        (idx, br, bc, _, semg, _) = s
        pltpu.async_copy(t_hbm.at[idx.at[0, 0]], br, semg)
        pltpu.async_copy(t_hbm.at[idx.at[0, 1]], bc, semg)

    def gathers_wait(s):
        (idx, br, bc, _, semg, _) = s
        pltpu.make_async_copy(t_hbm.at[idx.at[0, 0]], br, semg).wait()
        pltpu.make_async_copy(t_hbm.at[idx.at[0, 1]], bc, semg).wait()

    def out_start(g, s):
        (_, br, bc, _, _, semo) = s
        e0 = g * _CHUNK_E
        pltpu.async_copy(br, gr_hbm.at[pl.ds(e0, _CHUNK_E)], semo)
        pltpu.async_copy(bc, gc_hbm.at[pl.ds(e0, _CHUNK_E)], semo)

    def out_wait(g, s):
        (_, br, bc, _, _, semo) = s
        e0 = g * _CHUNK_E
        pltpu.make_async_copy(br, gr_hbm.at[pl.ds(e0, _CHUNK_E)], semo).wait()
        pltpu.make_async_copy(bc, gc_hbm.at[pl.ds(e0, _CHUNK_E)], semo).wait()

    idx_start(base, slots[0])
    idx_start(base + 1, slots[1])

    def round_body(i, carry):
        for b in (0, 1):
            t = 2 * i + b
            g = base + t
            s = slots[b]
            so = slots[1 - b]
            idx_wait(g, s)

            @pl.when(t >= 2)
            def _():
                out_wait(g - 2, s)

            gathers_start(s)

            @pl.when(t >= 1)
            def _():
                gathers_wait(so)
                out_start(g - 1, so)

            @pl.when((t >= 1) & (t + 1 < _CPW))
            def _():
                idx_start(g + 1, so)
        return carry

    lax.fori_loop(0, _NR, round_body, 0)
    last = base + _CPW - 1
    gathers_wait(slots[1])
    out_start(last, slots[1])
    out_wait(last - 1, slots[0])
    out_wait(last, slots[1])


@functools.cache
def _gather_kernel():
    return pl.kernel(
        _gather_body,
        out_type=(jax.ShapeDtypeStruct((_EP, _TW), jnp.float32),
                  jax.ShapeDtypeStruct((_EP, _TW), jnp.float32)),
        mesh=_sc_mesh(),
        scratch_types=(
            pltpu.VMEM((1, 2, 128), jnp.int32),
            pltpu.VMEM((1, 2, 128), jnp.int32),
            pltpu.VMEM((_CHUNK_E, _TW), jnp.float32),
            pltpu.VMEM((_CHUNK_E, _TW), jnp.float32),
            pltpu.VMEM((_CHUNK_E, _TW), jnp.float32),
            pltpu.VMEM((_CHUNK_E, _TW), jnp.float32),
            pltpu.SemaphoreType.DMA,
            pltpu.SemaphoreType.DMA,
            pltpu.SemaphoreType.DMA,
            pltpu.SemaphoreType.DMA,
            pltpu.SemaphoreType.DMA,
            pltpu.SemaphoreType.DMA,
        ),
    )


def _gather_call(t_tab, idx_all):
    return _gather_kernel()(t_tab, idx_all)


def _scatter_body(msg_hbm, cols_hbm, zeros_hbm, part_hbm,
                  acc, zbuf, idx0, idxm0, idx1, idxm1, bm0, bm1,
                  semi0, semi1, sems0, sems1):
    cid = lax.axis_index("c")
    sid = lax.axis_index("s")
    base = sid * _CPT
    lo = jnp.broadcast_to((cid * _HALF_N).astype(jnp.int32), (16,))
    hi = lo + _HALF_N
    dummy = jnp.full((16,), _HALF_N, jnp.int32)
    slots = ((idx0, idxm0, bm0, semi0, sems0),
             (idx1, idxm1, bm1, semi1, sems1))

    pltpu.sync_copy(zeros_hbm, zbuf)
    pltpu.sync_copy(zbuf, acc.at[pl.ds(sid * _TPR, _TPR)])
    plsc.subcore_barrier()

    def in_start(g, s):
        (idx, _, bm, semi, _) = s
        pltpu.async_copy(cols_hbm.at[pl.ds(g, 1)], idx, semi)
        pltpu.async_copy(msg_hbm.at[pl.ds(g * _CHUNK_E, _CHUNK_E)], bm, semi)

    def in_wait(g, s):
        (idx, _, bm, semi, _) = s
        pltpu.make_async_copy(cols_hbm.at[pl.ds(g, 1)], idx, semi).wait()
        pltpu.make_async_copy(msg_hbm.at[pl.ds(g * _CHUNK_E, _CHUNK_E)],
                              bm, semi).wait()

    def remap(s):
        (idx, idxm, _, _, _) = s
        for k in range(8):
            v = idx[0, pl.ds(k * 16, 16)]
            keep = (v >= lo) & (v < hi)
            idxm[0, pl.ds(k * 16, 16)] = jnp.where(keep, v - lo, dummy)

    def scat_start(s):
        (_, idxm, bm, _, sems) = s
        pltpu.async_copy(bm, acc.at[idxm.at[0]], sems, add=True)

    def scat_wait(s):
        (_, idxm, bm, _, sems) = s
        pltpu.make_async_copy(bm, acc.at[idxm.at[0]], sems).wait()

    in_start(base, slots[0])
    in_start(base + 1, slots[1])

    def round_body(i, carry):
        for b in (0, 1):
            t = 2 * i + b
            g = base + t
            s = slots[b]
            so = slots[1 - b]
            in_wait(g, s)
            remap(s)
            scat_start(s)

            @pl.when(t >= 1)
            def _():
                scat_wait(so)

            @pl.when((t >= 1) & (t + 1 < _CPT))
            def _():
                in_start(g + 1, so)
        return carry

    lax.fori_loop(0, _CPT // 2, round_body, 0)
    scat_wait(slots[1])
    plsc.subcore_barrier()

    pltpu.sync_copy(acc.at[pl.ds(sid * _TPR, _TPR)], zbuf)
    pltpu.sync_copy(zbuf, part_hbm.at[cid, pl.ds(sid * _TPR, _TPR)])


@functools.cache
def _scatter_kernel():
    return pl.kernel(
        _scatter_body,
        out_type=jax.ShapeDtypeStruct((_NC, _ACC_ROWS, _TW), jnp.float32),
        mesh=_sc_mesh(),
        scratch_types=(
            pltpu.VMEM_SHARED((_ACC_ROWS, _TW), jnp.float32),
            pltpu.VMEM((_TPR, _TW), jnp.float32),
            pltpu.VMEM((1, 128), jnp.int32),
            pltpu.VMEM((1, 128), jnp.int32),
            pltpu.VMEM((1, 128), jnp.int32),
            pltpu.VMEM((1, 128), jnp.int32),
            pltpu.VMEM((_CHUNK_E, _TW), jnp.float32),
            pltpu.VMEM((_CHUNK_E, _TW), jnp.float32),
            pltpu.SemaphoreType.DMA,
            pltpu.SemaphoreType.DMA,
            pltpu.SemaphoreType.DMA,
            pltpu.SemaphoreType.DMA,
        ),
    )


def _scatter_call(msg, cols_s, zeros_tile):
    return _scatter_kernel()(msg, cols_s, zeros_tile)


def _dot(x, w):
    return jnp.dot(x, w, preferred_element_type=jnp.float32)


_BLK_N = 1000


def _embed_body(nx, wn, bn, w1ab, tbias, nodes_out, t_out):
    n = _dot(nx[...], wn[...]) + bn[...]
    nodes_out[...] = n
    t_out[...] = _dot(n, w1ab[...]) + tbias[...]


def _embed_call(node_x, wn, bn, w1ab, tbias):
    full = lambda *shape: pl.BlockSpec(shape, lambda i: (0,) * len(shape))
    blk = lambda d: pl.BlockSpec((_BLK_N, d), lambda i: (i, 0))
    return pl.pallas_call(
        _embed_body,
        grid=(_N // _BLK_N,),
        in_specs=[blk(_RN), full(_RN, _NF), full(1, _NF),
                  full(_NF, _TW), full(1, _TW)],
        out_specs=[blk(_NF), blk(_TW)],
        out_shape=[jax.ShapeDtypeStruct((_N, _NF), jnp.float32),
                   jax.ShapeDtypeStruct((_N, _TW), jnp.float32)],
    )(node_x, wn, bn, w1ab, tbias)


_BLK_E = 2048


def _msg_body(gr, gc, ea, m, w2, b2, out):
    pre = gr[..., :_NF] + gc[..., _NF:] + _dot(ea[...], m[...])
    msg = _dot(jnp.maximum(pre, 0.0), w2[...]) + b2[...]
    out[...] = jnp.concatenate(
        [msg, jnp.zeros((_BLK_E, _TW - _NF), jnp.float32)], axis=1)


def _msg_call(grow, gcol, ea_pad, m, w2, b2):
    full = lambda *shape: pl.BlockSpec(shape, lambda i: (0,) * len(shape))
    blk = lambda d: pl.BlockSpec((_BLK_E, d), lambda i: (i, 0))
    return pl.pallas_call(
        _msg_body,
        grid=(_EP // _BLK_E,),
        in_specs=[blk(_TW), blk(_TW), blk(_RE),
                  full(_RE, _NF), full(_NF, _NF), full(1, _NF)],
        out_specs=blk(_TW),
        out_shape=jax.ShapeDtypeStruct((_EP, _TW), jnp.float32),
    )(grow, gcol, ea_pad, m, w2, b2)


def _update_body(nodes, p0, p1, u1a, u1b, ub1, u2, ub2,
                 w1ab, tbias, nodes_out, t_out):
    i = pl.program_id(0)
    agg = jnp.where(i < 5, p0[...], p1[...])[..., :_NF]
    h = jnp.maximum(_dot(nodes[...], u1a[...]) + _dot(agg, u1b[...])
                    + ub1[...], 0.0)
    n = _dot(h, u2[...]) + ub2[...]
    nodes_out[...] = n
    if t_out is not None:
        t_out[...] = _dot(n, w1ab[...]) + tbias[...]


def _update_call(nodes, p0, p1, u1a, u1b, ub1, u2, ub2, w1ab=None, tbias=None):
    full = lambda *shape: pl.BlockSpec(shape, lambda i: (0,) * len(shape))
    blk = lambda d: pl.BlockSpec((_BLK_N, d), lambda i: (i, 0))
    mid = w1ab is not None
    p_lo = pl.BlockSpec((_BLK_N, _TW), lambda i: (jnp.minimum(i, 4), 0))
    p_hi = pl.BlockSpec((_BLK_N, _TW), lambda i: (jnp.maximum(i - 5, 0), 0))
    in_specs = [blk(_NF), p_lo, p_hi,
                full(_NF, _NF), full(_NF, _NF), full(1, _NF),
                full(_NF, _NF), full(1, _NF)]
    args = [nodes, p0, p1, u1a, u1b, ub1, u2, ub2]
    if mid:
        in_specs += [full(_NF, _TW), full(1, _TW)]
        args += [w1ab, tbias]
        body = _update_body
        out_specs = [blk(_NF), blk(_TW)]
        out_shape = [jax.ShapeDtypeStruct((_N, _NF), jnp.float32),
                     jax.ShapeDtypeStruct((_N, _TW), jnp.float32)]
    else:
        body = lambda *refs: _update_body(*refs[:8], None, None, refs[8], None)
        out_specs = blk(_NF)
        out_shape = jax.ShapeDtypeStruct((_N, _NF), jnp.float32)
    return pl.pallas_call(
        body,
        grid=(_N // _BLK_N,),
        in_specs=in_specs,
        out_specs=out_specs,
        out_shape=out_shape,
    )(*args)


def _heads_body(gf, wg, bg, pw1, pb1, pw2, pb2, vw1, vb1, vw2, vb2,
                nodes, pol_out, val_out, rep_out):
    g = _dot(gf[...], wg[...]) + bg[...]
    ph = jnp.maximum(_dot(g, pw1[...]) + pb1[...], 0.0)
    pol_out[...] = _dot(ph, pw2[...]) + pb2[...]
    vh = jnp.maximum(_dot(g, vw1[...]) + vb1[...], 0.0)
    val_out[...] = jnp.tanh(_dot(vh, vw2[...]) + vb2[...])
    rep_out[...] = jnp.mean(nodes[...], axis=0, keepdims=True)


def _heads_call(gf, wg, bg, pw1, pb1, pw2, pb2, vw1, vb1, vw2, vb2, nodes):
    full = lambda *shape: pl.BlockSpec(shape, lambda: (0,) * len(shape))
    return pl.pallas_call(
        _heads_body,
        in_specs=[full(1, 100), full(100, _H), full(1, _H),
                  full(_H, _H), full(1, _H),
                  full(_H, _N), full(1, _N),
                  full(_H, 32), full(1, 32),
                  full(32, 1), full(1, 1),
                  full(_N, _NF)],
        out_specs=[full(1, _N), full(1, 1), full(1, _NF)],
        out_shape=[jax.ShapeDtypeStruct((1, _N), jnp.float32),
                   jax.ShapeDtypeStruct((1, 1), jnp.float32),
                   jax.ShapeDtypeStruct((1, _NF), jnp.float32)],
    )(gf, wg, bg, pw1, pb1, pw2, pb2, vw1, vb1, vw2, vb2, nodes)


def kernel(node_x, edge_index, edge_attr, global_feats, params):
    p = params
    f32 = jnp.float32

    row = edge_index[0]
    col = edge_index[1]
    pad0 = jnp.zeros((_EP - _E,), jnp.int32)
    rowp = jnp.concatenate([row, pad0]).reshape(_ROWS, 128)
    colp = jnp.concatenate([col, pad0]).reshape(_ROWS, 128)
    idx_all = jnp.stack([rowp, colp], axis=1)
    cols_s = jnp.concatenate(
        [col, jnp.full((_EP - _E,), _N, jnp.int32)]).reshape(_ROWS, 128)
    ea_pad = jnp.concatenate(
        [edge_attr, jnp.zeros((_EP - _E, _RE), f32)], axis=0)
    zeros_tile = jnp.zeros((_TPR, _TW), f32)
    row2 = lambda v: v.reshape(1, -1)

    layers = p['layers']
    w1ab, m_l, u1a, u1b, ub1, u2, ub2 = [], [], [], [], [], [], []
    tbias = []
    for lp in layers:
        w1 = lp['msg_w1']
        w1ab.append(jnp.concatenate([w1[:_NF], w1[_NF:2 * _NF]], axis=1))
        w1c = w1[2 * _NF:]
        m_l.append(p['edge_emb_w'] @ w1c)
        tbias.append(row2(jnp.concatenate(
            [jnp.zeros((_NF,), f32),
             lp['msg_b1'] + p['edge_emb_b'] @ w1c])))
        u1 = lp['upd_w1']
        u1a.append(u1[:_NF])
        u1b.append(u1[_NF:])
        ub1.append(row2(lp['upd_b1']))
        u2.append(lp['upd_w2'])
        ub2.append(row2(lp['upd_b2']))

    nodes, t_tab = _embed_call(
        node_x, p['node_emb_w'], row2(p['node_emb_b']), w1ab[0], tbias[0])

    for l in range(len(layers)):
        grow, gcol = _gather_call(t_tab, idx_all)
        msg = _msg_call(grow, gcol, ea_pad, m_l[l],
                        layers[l]['msg_w2'], row2(layers[l]['msg_b2']))
        part = _scatter_call(msg, cols_s, zeros_tile)
        p0 = part[0, :_HALF_N]
        p1 = part[1, :_HALF_N]
        if l + 1 < len(layers):
            nodes, t_tab = _update_call(
                nodes, p0, p1, u1a[l], u1b[l], ub1[l], u2[l], ub2[l],
                w1ab[l + 1], tbias[l + 1])
        else:
            nodes = _update_call(
                nodes, p0, p1, u1a[l], u1b[l], ub1[l], u2[l], ub2[l])

    policy, value, rep = _heads_call(
        global_feats.reshape(1, 100), p['glob_w'], row2(p['glob_b']),
        p['pol_w1'], row2(p['pol_b1']), p['pol_w2'], row2(p['pol_b2']),
        p['val_w1'], row2(p['val_b1']), p['val_w2'], row2(p['val_b2']),
        nodes)
    return (policy.reshape(_N), value.reshape(1), rep.reshape(_NF))
